# baseline (speedup 1.0000x reference)
import functools

import jax
import jax.numpy as jnp
from jax import lax
from jax.experimental import pallas as pl
from jax.experimental.pallas import tpu as pltpu
from jax.experimental.pallas import tpu_sc as plsc

EMBED = 32
TOKENS_PER_ROW = 26 * 50
CHUNK = 128
NCHUNK = 11
PAD_TOKENS = CHUNK * NCHUNK
BATCH = 4096
ROWS_PER_W = BATCH // 32


def _sc_pool_body(tok_hbm, table_hbm, out_hbm, idx0, idx1, rows0, rows1,
                  pool_v, sem0, sem1):
  nc = 2
  wid = lax.axis_index("s") * nc + lax.axis_index("c")
  base = wid * ROWS_PER_W

  zero16 = jnp.zeros((16,), jnp.float32)

  def fire(r, idx_v, rows_v, sem):
    pltpu.sync_copy(tok_hbm.at[base + r], idx_v)
    for c in range(NCHUNK - 1):
      pltpu.async_copy(table_hbm.at[idx_v.at[c]],
                       rows_v.at[pl.ds(c * CHUNK, CHUNK)], sem)
    tail = TOKENS_PER_ROW - (NCHUNK - 1) * CHUNK
    pltpu.async_copy(table_hbm.at[idx_v.at[NCHUNK - 1].at[pl.ds(0, tail)]],
                     rows_v.at[pl.ds((NCHUNK - 1) * CHUNK, tail)], sem)

  def drain(rows_v, sem):
    pltpu.make_async_copy(out_hbm.at[pl.ds(0, TOKENS_PER_ROW)],
                          rows_v.at[pl.ds(0, TOKENS_PER_ROW)], sem).wait()

  def accum(r, rows_v):
    def tok_body(t, carry):
      a0, a1 = carry
      a0 = a0 + rows_v[t, pl.ds(0, 16)]
      a1 = a1 + rows_v[t, pl.ds(16, 16)]
      return (a0, a1)

    a0, a1 = lax.fori_loop(0, TOKENS_PER_ROW, tok_body, (zero16, zero16),
                           unroll=10)
    pool_v[r, pl.ds(0, 16)] = a0
    pool_v[r, pl.ds(16, 16)] = a1

  fire(0, idx0, rows0, sem0)

  def pair_body(g, _):
    fire(2 * g + 1, idx1, rows1, sem1)
    drain(rows0, sem0)
    accum(2 * g, rows0)

    @pl.when(g < ROWS_PER_W // 2 - 1)
    def _():
      fire(2 * g + 2, idx0, rows0, sem0)

    drain(rows1, sem1)
    accum(2 * g + 1, rows1)
    return 0

  lax.fori_loop(0, ROWS_PER_W // 2, pair_body, 0)
  pltpu.sync_copy(pool_v, out_hbm.at[pl.ds(base, ROWS_PER_W)])


_sc_pool = functools.partial(
    pl.kernel,
    out_type=jax.ShapeDtypeStruct((BATCH, EMBED), jnp.float32),
    mesh=plsc.VectorSubcoreMesh(core_axis_name="c", subcore_axis_name="s"),
    compiler_params=pltpu.CompilerParams(use_tc_tiling_on_sc=False),
    scratch_types=[
        pltpu.VMEM((NCHUNK, CHUNK), jnp.int32),
        pltpu.VMEM((NCHUNK, CHUNK), jnp.int32),
        pltpu.VMEM((PAD_TOKENS, EMBED), jnp.float32),
        pltpu.VMEM((PAD_TOKENS, EMBED), jnp.float32),
        pltpu.VMEM((ROWS_PER_W, EMBED), jnp.float32),
        pltpu.SemaphoreType.DMA,
        pltpu.SemaphoreType.DMA,
    ],
)(_sc_pool_body)


def _mlp_body(s_ref, w1_ref, b1_ref, w2_ref, b2_ref, o_ref):
  x = s_ref[...] * (1.0 / TOKENS_PER_ROW)
  h = lax.dot_general(x, w1_ref[...], (((1,), (1,)), ((), ())),
                      preferred_element_type=jnp.float32)
  h = jnp.maximum(h + b1_ref[...], 0.0)
  o_ref[...] = jnp.sum(h * w2_ref[...], axis=1, keepdims=True) + b2_ref[...]


def _tc_mlp(sums, w1, b1, w2, b2):
  return pl.pallas_call(
      _mlp_body,
      out_shape=jax.ShapeDtypeStruct((BATCH, 1), jnp.float32),
  )(sums, w1, b1.reshape(1, EMBED), w2, b2.reshape(1, 1))


@jax.jit
def kernel(tokens, table, W1, b1, W2, b2):
  tok = tokens.reshape(BATCH, TOKENS_PER_ROW).astype(jnp.int32)
  tok = jnp.pad(tok, ((0, 0), (0, PAD_TOKENS - TOKENS_PER_ROW)))
  sums = _sc_pool(tok.reshape(BATCH, NCHUNK, CHUNK), table)
  out = _tc_mlp(sums, W1, b1, W2, b2)
  return out.reshape(BATCH)


# speedup vs baseline: 1.0462x; 1.0462x over previous
import functools

import jax
import jax.numpy as jnp
from jax import lax
from jax.experimental import pallas as pl
from jax.experimental.pallas import tpu as pltpu
from jax.experimental.pallas import tpu_sc as plsc

EMBED = 32
NFEAT = 26
SLOTS = 50
TOKENS_PER_ROW = NFEAT * SLOTS
BATCH = 4096
ROWS_PER_W = BATCH // 32


def _sc_pool_body(tok_hbm, table_hbm, zdummy_hbm, out_hbm, idx0, idx1,
                  rows0, rows1, pool_v, sem0, sem1, semi):
  nc = 2
  wid = lax.axis_index("s") * nc + lax.axis_index("c")
  base = wid * ROWS_PER_W

  zero16 = jnp.zeros((16,), jnp.float32)

  def stage(r, idx_v):
    pltpu.async_copy(tok_hbm.at[base + r], idx_v, semi)

  def drain_idx(idx_v):
    pltpu.make_async_copy(zdummy_hbm, idx_v, semi).wait()

  def fire(idx_v, rows_v, sem):
    def feat_body(f, _):
      pltpu.async_copy(table_hbm.at[idx_v.at[f]],
                       rows_v.at[pl.ds(f * SLOTS, SLOTS)], sem)
      return 0

    lax.fori_loop(0, NFEAT, feat_body, 0)

  def drain_rows(rows_v, sem):
    pltpu.make_async_copy(out_hbm.at[pl.ds(0, TOKENS_PER_ROW)], rows_v,
                          sem).wait()

  def accum(r, rows_v):
    def tok_body(t, carry):
      a0, a1 = carry
      a0 = a0 + rows_v[t, pl.ds(0, 16)]
      a1 = a1 + rows_v[t, pl.ds(16, 16)]
      return (a0, a1)

    a0, a1 = lax.fori_loop(0, TOKENS_PER_ROW, tok_body, (zero16, zero16),
                           unroll=10)
    pool_v[r, pl.ds(0, 16)] = a0
    pool_v[r, pl.ds(16, 16)] = a1

  stage(0, idx0)
  drain_idx(idx0)
  fire(idx0, rows0, sem0)
  stage(1, idx1)

  def pair_body(g, _):
    rn2 = jnp.minimum(2 * g + 2, ROWS_PER_W - 1)
    rn3 = jnp.minimum(2 * g + 3, ROWS_PER_W - 1)

    drain_idx(idx1)
    fire(idx1, rows1, sem1)

    drain_rows(rows0, sem0)
    stage(rn2, idx0)
    accum(2 * g, rows0)

    drain_idx(idx0)
    fire(idx0, rows0, sem0)

    drain_rows(rows1, sem1)
    stage(rn3, idx1)
    accum(2 * g + 1, rows1)
    return 0

  lax.fori_loop(0, ROWS_PER_W // 2, pair_body, 0)
  drain_idx(idx1)
  drain_rows(rows0, sem0)
  pltpu.sync_copy(pool_v, out_hbm.at[pl.ds(base, ROWS_PER_W)])


_sc_pool = functools.partial(
    pl.kernel,
    out_type=jax.ShapeDtypeStruct((BATCH, EMBED), jnp.float32),
    mesh=plsc.VectorSubcoreMesh(core_axis_name="c", subcore_axis_name="s"),
    compiler_params=pltpu.CompilerParams(use_tc_tiling_on_sc=False),
    scratch_types=[
        pltpu.VMEM((NFEAT, SLOTS), jnp.int32),
        pltpu.VMEM((NFEAT, SLOTS), jnp.int32),
        pltpu.VMEM((TOKENS_PER_ROW, EMBED), jnp.float32),
        pltpu.VMEM((TOKENS_PER_ROW, EMBED), jnp.float32),
        pltpu.VMEM((ROWS_PER_W, EMBED), jnp.float32),
        pltpu.SemaphoreType.DMA,
        pltpu.SemaphoreType.DMA,
        pltpu.SemaphoreType.DMA,
    ],
)(_sc_pool_body)


def _mlp_body(s_ref, w1_ref, b1_ref, w2_ref, b2_ref, o_ref):
  x = s_ref[...] * (1.0 / TOKENS_PER_ROW)
  h = lax.dot_general(x, w1_ref[...], (((1,), (1,)), ((), ())),
                      preferred_element_type=jnp.float32)
  h = jnp.maximum(h + b1_ref[...], 0.0)
  o_ref[...] = jnp.sum(h * w2_ref[...], axis=1, keepdims=True) + b2_ref[...]


def _tc_mlp(sums, w1, b1, w2, b2):
  return pl.pallas_call(
      _mlp_body,
      out_shape=jax.ShapeDtypeStruct((BATCH, 1), jnp.float32),
  )(sums, w1, b1.reshape(1, EMBED), w2, b2.reshape(1, 1))


@jax.jit
def kernel(tokens, table, W1, b1, W2, b2):
  tok = tokens.astype(jnp.int32)
  zdummy = jnp.zeros((NFEAT, SLOTS), jnp.int32)
  sums = _sc_pool(tok, table, zdummy)
  out = _tc_mlp(sums, W1, b1, W2, b2)
  return out.reshape(BATCH)
